# Initial kernel scaffold; baseline (speedup 1.0000x reference)
import jax
import jax.numpy as jnp
from jax.experimental import pallas as pl


def kernel(x, weights):
    raise NotImplementedError("write your pallas kernel here")


# speedup vs baseline: 1.1025x; 1.1025x over previous
import functools

import jax
import jax.numpy as jnp
from jax import lax
from jax.experimental import pallas as pl
from jax.experimental.pallas import tpu as pltpu
from jax.experimental.pallas import tpu_sc as plsc

D_MODEL = 32
NUM_CORES = 2
NUM_SUBCORES = 16
NUM_WORKERS = NUM_CORES * NUM_SUBCORES


@functools.lru_cache(maxsize=None)
def _make_emb_kernel(n_rows: int, chunk: int):
    rows_per_w = n_rows // NUM_WORKERS
    n_chunks = rows_per_w // chunk
    mesh = plsc.VectorSubcoreMesh(core_axis_name="c", subcore_axis_name="s")

    @functools.partial(
        pl.kernel,
        mesh=mesh,
        out_type=jax.ShapeDtypeStruct((n_rows, D_MODEL), jnp.float32),
        scratch_types=[
            pltpu.VMEM((chunk,), jnp.int32),
            pltpu.VMEM((chunk, D_MODEL), jnp.float32),
            pltpu.SemaphoreType.DMA,
        ],
        compiler_params=pltpu.CompilerParams(use_tc_tiling_on_sc=False),
    )
    def emb(idx_hbm, table_hbm, out_hbm, idx_v, rows_v, sem):
        wid = lax.axis_index("s") * NUM_CORES + lax.axis_index("c")
        base = wid * rows_per_w

        def body(c, carry):
            off = base + c * chunk
            pltpu.sync_copy(idx_hbm.at[pl.ds(off, chunk)], idx_v)
            pltpu.async_copy(table_hbm.at[idx_v], rows_v, sem).wait()
            pltpu.sync_copy(rows_v, out_hbm.at[pl.ds(off, chunk)])
            return carry

        lax.fori_loop(0, n_chunks, body, 0)

    return emb


def kernel(x, weights):
    b, l = x.shape
    flat_idx = x.reshape(b * l).astype(jnp.int32)
    emb = _make_emb_kernel(b * l, 1600)
    out = emb(flat_idx, weights)
    return out.reshape(b, l, D_MODEL)


# speedup vs baseline: 1.1092x; 1.0061x over previous
import functools

import jax
import jax.numpy as jnp
from jax import lax
from jax.experimental import pallas as pl
from jax.experimental.pallas import tpu as pltpu
from jax.experimental.pallas import tpu_sc as plsc

D_MODEL = 32
NUM_CORES = 2
NUM_SUBCORES = 16
NUM_WORKERS = NUM_CORES * NUM_SUBCORES


@functools.lru_cache(maxsize=None)
def _make_emb_kernel(n_rows: int, chunk: int):
    rows_per_w = n_rows // NUM_WORKERS
    n_chunks = rows_per_w // chunk
    assert n_chunks % 2 == 0 and n_chunks >= 2
    mesh = plsc.VectorSubcoreMesh(core_axis_name="c", subcore_axis_name="s")

    @functools.partial(
        pl.kernel,
        mesh=mesh,
        out_type=jax.ShapeDtypeStruct((n_rows, D_MODEL), jnp.float32),
        scratch_types=[
            pltpu.VMEM((chunk,), jnp.int32),
            pltpu.VMEM((chunk,), jnp.int32),
            pltpu.VMEM((chunk, D_MODEL), jnp.float32),
            pltpu.VMEM((chunk, D_MODEL), jnp.float32),
            pltpu.SemaphoreType.DMA,
            pltpu.SemaphoreType.DMA,
            pltpu.SemaphoreType.DMA,
            pltpu.SemaphoreType.DMA,
        ],
        compiler_params=pltpu.CompilerParams(use_tc_tiling_on_sc=False),
    )
    def emb(idx_hbm, table_hbm, out_hbm, idx0, idx1, rows0, rows1,
            isem0, isem1, gsem0, gsem1):
        wid = lax.axis_index("s") * NUM_CORES + lax.axis_index("c")
        base = wid * rows_per_w
        idx_v = (idx0, idx1)
        rows_v = (rows0, rows1)
        isem = (isem0, isem1)
        gsem = (gsem0, gsem1)

        def idx_load(c, b):
            off = base + c * chunk
            pltpu.async_copy(idx_hbm.at[pl.ds(off, chunk)], idx_v[b], isem[b])

        def idx_wait(c, b):
            off = base + c * chunk
            pltpu.make_async_copy(
                idx_hbm.at[pl.ds(off, chunk)], idx_v[b], isem[b]).wait()

        def gather_start(b):
            pltpu.async_copy(table_hbm.at[idx_v[b]], rows_v[b], gsem[b])

        def gather_wait(b):
            pltpu.make_async_copy(
                table_hbm.at[idx_v[b]], rows_v[b], gsem[b]).wait()

        idx_load(0, 0)
        idx_load(1, 1)
        idx_wait(0, 0)
        gather_start(0)

        def group(g, carry):
            for b in range(2):
                c = g * 2 + b
                o = 1 - b
                gather_wait(b)

                @pl.when(c < n_chunks - 2)
                def _():
                    idx_load(c + 2, b)

                @pl.when(c < n_chunks - 1)
                def _():
                    idx_wait(c + 1, o)
                    gather_start(o)

                pltpu.sync_copy(rows_v[b],
                                out_hbm.at[pl.ds(base + c * chunk, chunk)])
            return carry

        lax.fori_loop(0, n_chunks // 2, group, 0)

    return emb


def kernel(x, weights):
    b, l = x.shape
    flat_idx = x.reshape(b * l).astype(jnp.int32)
    emb = _make_emb_kernel(b * l, 1600)
    out = emb(flat_idx, weights)
    return out.reshape(b, l, D_MODEL)


# speedup vs baseline: 1.5742x; 1.4193x over previous
import functools

import jax
import jax.numpy as jnp
from jax import lax
from jax.experimental import pallas as pl
from jax.experimental.pallas import tpu as pltpu
from jax.experimental.pallas import tpu_sc as plsc

D_MODEL = 32
NUM_CORES = 2
NUM_SUBCORES = 16
NUM_WORKERS = NUM_CORES * NUM_SUBCORES

B = 16384
L = 50
BT = 128
N_BT = B // BT
BT_PER_W = N_BT // NUM_WORKERS
L_GRP = 10
N_CHUNK = L // L_GRP
CHUNK_ROWS = L_GRP * BT
N_SLAB = D_MODEL // 8


@functools.lru_cache(maxsize=None)
def _make_emb_kernel():
    mesh = plsc.VectorSubcoreMesh(core_axis_name="c", subcore_axis_name="s")

    @functools.partial(
        pl.kernel,
        mesh=mesh,
        out_type=jax.ShapeDtypeStruct((L, N_SLAB, N_BT, 8, BT), jnp.float32),
        scratch_types=[
            pltpu.VMEM((BT * L,), jnp.int32),
            pltpu.VMEM((CHUNK_ROWS,), jnp.int32),
            pltpu.VMEM((CHUNK_ROWS, D_MODEL), jnp.float32),
            pltpu.VMEM((L_GRP, N_SLAB, 8, BT), jnp.float32),
            pltpu.SemaphoreType.DMA,
            pltpu.SemaphoreType.DMA,
        ],
        compiler_params=pltpu.CompilerParams(
            use_tc_tiling_on_sc=False, needs_layout_passes=False),
    )
    def emb(idx_hbm, table_hbm, out_hbm, xb, idxp, rows, tbuf, gsem, osem):
        wid = lax.axis_index("s") * NUM_CORES + lax.axis_index("c")
        iota16 = lax.iota(jnp.int32, 16)

        def unit(u, carry):
            bti = u // N_CHUNK
            ch = u % N_CHUNK
            bt_abs = wid * BT_PER_W + bti
            tok0 = bt_abs * (BT * L)
            l0 = ch * L_GRP

            @pl.when(ch == 0)
            def _():
                pltpu.sync_copy(idx_hbm.at[pl.ds(tok0, BT * L)], xb)

            for g in range(CHUNK_ROWS // 16):
                p0 = g * 16
                ll = p0 // BT
                c0 = p0 % BT
                src = (c0 + iota16) * L + (l0 + ll)
                idxp[pl.ds(p0, 16)] = plsc.load_gather(xb, [src])

            pltpu.async_copy(table_hbm.at[idxp], rows, gsem).wait()

            def tile_body(t, tc):
                ll = t // N_SLAB
                s = t % N_SLAB
                for d8 in range(8):
                    col = jnp.full((16,), s * 8 + d8, jnp.int32)
                    for g2 in range(BT // 16):
                        c0 = g2 * 16
                        rowi = ll * BT + c0 + iota16
                        tbuf[ll, s, d8, pl.ds(c0, 16)] = plsc.load_gather(
                            rows, [rowi, col])
                pltpu.async_copy(tbuf.at[ll, s],
                                 out_hbm.at[l0 + ll, s, bt_abs], osem)
                return tc

            lax.fori_loop(0, L_GRP * N_SLAB, tile_body, 0)

            def drain(t, tc):
                ll = t // N_SLAB
                s = t % N_SLAB
                pltpu.make_async_copy(
                    tbuf.at[ll, s], out_hbm.at[l0 + ll, s, bt_abs],
                    osem).wait()
                return tc

            lax.fori_loop(0, L_GRP * N_SLAB, drain, 0)
            return carry

        lax.fori_loop(0, BT_PER_W * N_CHUNK, unit, 0)

    return emb


def kernel(x, weights):
    b, l = x.shape
    flat_idx = x.reshape(b * l).astype(jnp.int32)
    emb = _make_emb_kernel()
    out5 = emb(flat_idx, weights)
    return jnp.transpose(out5, (2, 4, 0, 1, 3)).reshape(b, l, D_MODEL)


# speedup vs baseline: 1.8380x; 1.1675x over previous
import functools

import jax
import jax.numpy as jnp
from jax import lax
from jax.experimental import pallas as pl
from jax.experimental.pallas import tpu as pltpu
from jax.experimental.pallas import tpu_sc as plsc

D_MODEL = 32
NUM_CORES = 2
NUM_SUBCORES = 16
NUM_WORKERS = NUM_CORES * NUM_SUBCORES

B = 16384
L = 50
BT = 128
N_BT = B // BT
BT_PER_W = N_BT // NUM_WORKERS
L_GRP = 10
N_CHUNK = L // L_GRP
CHUNK_ROWS = L_GRP * BT
N_SLAB = D_MODEL // 8


@functools.lru_cache(maxsize=None)
def _make_emb_kernel():
    mesh = plsc.VectorSubcoreMesh(core_axis_name="c", subcore_axis_name="s")

    @functools.partial(
        pl.kernel,
        mesh=mesh,
        out_type=jax.ShapeDtypeStruct((L, N_SLAB, N_BT, 8, BT), jnp.float32),
        scratch_types=[
            pltpu.VMEM((BT * L,), jnp.int32),
            pltpu.VMEM((CHUNK_ROWS,), jnp.int32),
            pltpu.VMEM((CHUNK_ROWS, D_MODEL), jnp.float32),
            pltpu.VMEM((L_GRP, N_SLAB, 8, BT), jnp.float32),
            pltpu.SemaphoreType.DMA,
            pltpu.SemaphoreType.DMA,
        ],
        compiler_params=pltpu.CompilerParams(
            use_tc_tiling_on_sc=False, needs_layout_passes=False),
    )
    def emb(idx_hbm, table_hbm, out_hbm, xb, idxp, rows, tbuf, gsem, osem):
        wid = lax.axis_index("s") * NUM_CORES + lax.axis_index("c")
        iota16 = lax.iota(jnp.int32, 16)
        iota_l = iota16 * L

        def unit(u, carry):
            bti = u // N_CHUNK
            ch = u % N_CHUNK
            bt_abs = wid * BT_PER_W + bti
            tok0 = bt_abs * (BT * L)
            l0 = ch * L_GRP

            @pl.when(ch == 0)
            def _():
                pltpu.sync_copy(idx_hbm.at[pl.ds(tok0, BT * L)], xb)

            @plsc.parallel_loop(0, CHUNK_ROWS // 16, unroll=8)
            def _(g):
                ll = g >> 3
                c0 = (g & 7) * 16
                src = iota_l + (c0 * L + l0 + ll)
                idxp[pl.ds(g * 16, 16)] = plsc.load_gather(xb, [src])

            pltpu.async_copy(table_hbm.at[idxp], rows, gsem).wait()

            @plsc.parallel_loop(0, L_GRP * N_SLAB * 8 * (BT // 16), unroll=8)
            def _(g):
                ll = g >> 8
                s = (g >> 6) & 3
                d8 = (g >> 3) & 7
                c0 = (g & 7) * 16
                rowi = iota16 + (ll * BT + c0)
                coli = jnp.full((16,), 0, jnp.int32) + (s * 8 + d8)
                tbuf[ll, s, d8, pl.ds(c0, 16)] = plsc.load_gather(
                    rows, [rowi, coli])

            pltpu.async_copy(
                tbuf, out_hbm.at[pl.ds(l0, L_GRP), :, bt_abs], osem).wait()
            return carry

        lax.fori_loop(0, BT_PER_W * N_CHUNK, unit, 0)

    return emb


def kernel(x, weights):
    b, l = x.shape
    flat_idx = x.reshape(b * l).astype(jnp.int32)
    emb = _make_emb_kernel()
    out5 = emb(flat_idx, weights)
    return jnp.transpose(out5, (2, 4, 0, 1, 3)).reshape(b, l, D_MODEL)


# speedup vs baseline: 1.9323x; 1.0513x over previous
import functools

import jax
import jax.numpy as jnp
from jax import lax
from jax.experimental import pallas as pl
from jax.experimental.pallas import tpu as pltpu
from jax.experimental.pallas import tpu_sc as plsc

D_MODEL = 32
NUM_CORES = 2
NUM_SUBCORES = 16
NUM_WORKERS = NUM_CORES * NUM_SUBCORES

B = 16384
L = 50
BT = 128
N_BT = B // BT
BT_PER_W = N_BT // NUM_WORKERS
L_GRP = 5
N_CHUNK = L // L_GRP
CHUNK_ROWS = L_GRP * BT
N_SLAB = D_MODEL // 8
N_UNIT = BT_PER_W * N_CHUNK


@functools.lru_cache(maxsize=None)
def _make_emb_kernel():
    mesh = plsc.VectorSubcoreMesh(core_axis_name="c", subcore_axis_name="s")

    @functools.partial(
        pl.kernel,
        mesh=mesh,
        out_type=jax.ShapeDtypeStruct((L, N_SLAB, N_BT, 8, BT), jnp.float32),
        scratch_types=[
            pltpu.VMEM((BT * L,), jnp.int32),
            pltpu.VMEM((CHUNK_ROWS,), jnp.int32),
            pltpu.VMEM((CHUNK_ROWS,), jnp.int32),
            pltpu.VMEM((CHUNK_ROWS, D_MODEL), jnp.float32),
            pltpu.VMEM((CHUNK_ROWS, D_MODEL), jnp.float32),
            pltpu.VMEM((L_GRP, N_SLAB, 8, BT), jnp.float32),
            pltpu.SemaphoreType.DMA,
            pltpu.SemaphoreType.DMA,
            pltpu.SemaphoreType.DMA,
        ],
        compiler_params=pltpu.CompilerParams(
            use_tc_tiling_on_sc=False, needs_layout_passes=False),
    )
    def emb(idx_hbm, table_hbm, out_hbm, xb, idxp0, idxp1, rows0, rows1,
            tbuf, gsem0, gsem1, osem):
        wid = lax.axis_index("s") * NUM_CORES + lax.axis_index("c")
        iota16 = lax.iota(jnp.int32, 16)
        iota_l = iota16 * L
        idxp = (idxp0, idxp1)
        rows = (rows0, rows1)
        gsem = (gsem0, gsem1)

        def stage_x(u):
            bt_abs = wid * BT_PER_W + u // N_CHUNK
            pltpu.sync_copy(idx_hbm.at[pl.ds(bt_abs * (BT * L), BT * L)], xb)

        def idx_build(u, slot):
            l0 = (u % N_CHUNK) * L_GRP

            @plsc.parallel_loop(0, CHUNK_ROWS // 16, unroll=8)
            def _(g):
                ll = g >> 3
                c0 = (g & 7) * 16
                src = iota_l + (c0 * L + l0 + ll)
                idxp[slot][pl.ds(g * 16, 16)] = plsc.load_gather(xb, [src])

        def gather_start(slot):
            pltpu.async_copy(table_hbm.at[idxp[slot]], rows[slot], gsem[slot])

        def gather_wait(slot):
            pltpu.make_async_copy(
                table_hbm.at[idxp[slot]], rows[slot], gsem[slot]).wait()

        def transpose(slot):
            @plsc.parallel_loop(0, L_GRP * N_SLAB * 8 * (BT // 16), unroll=8)
            def _(g):
                ll = g >> 8
                s = (g >> 6) & 3
                d8 = (g >> 3) & 7
                c0 = (g & 7) * 16
                rowi = iota16 + (ll * BT + c0)
                coli = jnp.full((16,), 0, jnp.int32) + (s * 8 + d8)
                tbuf[ll, s, d8, pl.ds(c0, 16)] = plsc.load_gather(
                    rows[slot], [rowi, coli])

        def out_slice(u):
            bt_abs = wid * BT_PER_W + u // N_CHUNK
            l0 = (u % N_CHUNK) * L_GRP
            return out_hbm.at[pl.ds(l0, L_GRP), :, bt_abs]

        stage_x(0)
        idx_build(0, 0)
        gather_start(0)

        def group(gidx, carry):
            for b in range(2):
                o = 1 - b
                u = gidx * 2 + b

                @pl.when(jnp.logical_and(u < N_UNIT - 1,
                                         (u + 1) % N_CHUNK == 0))
                def _():
                    stage_x(u + 1)

                gather_wait(b)

                @pl.when(u < N_UNIT - 1)
                def _():
                    idx_build(u + 1, o)
                    gather_start(o)

                @pl.when(u >= 1)
                def _():
                    pltpu.make_async_copy(tbuf, out_slice(u), osem).wait()

                transpose(b)
                pltpu.async_copy(tbuf, out_slice(u), osem)
            return carry

        lax.fori_loop(0, N_UNIT // 2, group, 0)
        pltpu.make_async_copy(tbuf, out_slice(N_UNIT - 1), osem).wait()

    return emb


def kernel(x, weights):
    b, l = x.shape
    flat_idx = x.reshape(b * l).astype(jnp.int32)
    emb = _make_emb_kernel()
    out5 = emb(flat_idx, weights)
    return jnp.transpose(out5, (2, 4, 0, 1, 3)).reshape(b, l, D_MODEL)


# speedup vs baseline: 3.1182x; 1.6138x over previous
import functools

import jax
import jax.numpy as jnp
from jax import lax
from jax.experimental import pallas as pl
from jax.experimental.pallas import tpu as pltpu
from jax.experimental.pallas import tpu_sc as plsc

D_MODEL = 32
NUM_CORES = 2
NUM_SUBCORES = 16
NUM_WORKERS = NUM_CORES * NUM_SUBCORES

B = 16384
L = 50
BT = 128
N_BT = B // BT
BT_PER_W = N_BT // NUM_WORKERS
L_GRP = 5
N_CHUNK = L // L_GRP
CHUNK_ROWS = L_GRP * BT
N_SLAB = D_MODEL // 8
N_UNIT = BT_PER_W * N_CHUNK
TPITCH = 129


@functools.lru_cache(maxsize=None)
def _make_emb_kernel():
    mesh = plsc.VectorSubcoreMesh(core_axis_name="c", subcore_axis_name="s")

    @functools.partial(
        pl.kernel,
        mesh=mesh,
        out_type=jax.ShapeDtypeStruct((L, N_SLAB, N_BT, 8, BT), jnp.float32),
        scratch_types=[
            pltpu.VMEM((BT * L,), jnp.int32),
            pltpu.VMEM((CHUNK_ROWS,), jnp.int32),
            pltpu.VMEM((CHUNK_ROWS,), jnp.int32),
            pltpu.VMEM((CHUNK_ROWS, D_MODEL), jnp.float32),
            pltpu.VMEM((CHUNK_ROWS, D_MODEL), jnp.float32),
            pltpu.VMEM((L_GRP * D_MODEL, TPITCH), jnp.float32),
            pltpu.SemaphoreType.DMA,
            pltpu.SemaphoreType.DMA,
            pltpu.SemaphoreType.DMA,
        ],
        compiler_params=pltpu.CompilerParams(
            use_tc_tiling_on_sc=False, needs_layout_passes=False),
    )
    def emb(idx_hbm, table_hbm, out_hbm, xb, idxp0, idxp1, rows0, rows1,
            tbuf, gsem0, gsem1, osem):
        wid = lax.axis_index("s") * NUM_CORES + lax.axis_index("c")
        iota16 = lax.iota(jnp.int32, 16)
        iota_l = iota16 * L
        zeros16 = jnp.full((16,), 0, jnp.int32)
        idxp = (idxp0, idxp1)
        rows = (rows0, rows1)
        gsem = (gsem0, gsem1)

        def stage_x(u):
            bt_abs = wid * BT_PER_W + u // N_CHUNK
            pltpu.sync_copy(idx_hbm.at[pl.ds(bt_abs * (BT * L), BT * L)], xb)

        def idx_build(u, slot):
            l0 = (u % N_CHUNK) * L_GRP

            @plsc.parallel_loop(0, CHUNK_ROWS // 16, unroll=8)
            def _(g):
                ll = g >> 3
                c0 = (g & 7) * 16
                src = iota_l + (c0 * L + l0 + ll)
                idxp[slot][pl.ds(g * 16, 16)] = plsc.load_gather(xb, [src])

        def gather_start(slot):
            pltpu.async_copy(table_hbm.at[idxp[slot]], rows[slot], gsem[slot])

        def gather_wait(slot):
            pltpu.make_async_copy(
                table_hbm.at[idxp[slot]], rows[slot], gsem[slot]).wait()

        def transpose(slot):
            @plsc.parallel_loop(0, CHUNK_ROWS * 2, unroll=8)
            def _(g):
                rp = g >> 1
                half = g & 1
                ll = rp >> 7
                c = rp & 127
                v = rows[slot][rp, pl.ds(half * 16, 16)]
                rowvec = iota16 + (ll * D_MODEL + half * 16)
                colvec = zeros16 + c
                plsc.store_scatter(tbuf, [rowvec, colvec], v)

        def out_tiles(u, start):
            bt_abs = wid * BT_PER_W + u // N_CHUNK
            l0 = (u % N_CHUNK) * L_GRP
            for ll in range(L_GRP):
                for s in range(N_SLAB):
                    src = tbuf.at[pl.ds(ll * D_MODEL + s * 8, 8),
                                  pl.ds(0, BT)]
                    dst = out_hbm.at[l0 + ll, s, bt_abs]
                    if start:
                        pltpu.async_copy(src, dst, osem)
                    else:
                        pltpu.make_async_copy(src, dst, osem).wait()

        stage_x(0)
        idx_build(0, 0)
        gather_start(0)

        def group(gidx, carry):
            for b in range(2):
                o = 1 - b
                u = gidx * 2 + b

                @pl.when(jnp.logical_and(u < N_UNIT - 1,
                                         (u + 1) % N_CHUNK == 0))
                def _():
                    stage_x(u + 1)

                gather_wait(b)

                @pl.when(u < N_UNIT - 1)
                def _():
                    idx_build(u + 1, o)
                    gather_start(o)

                @pl.when(u >= 1)
                def _():
                    out_tiles(u - 1, start=False)

                transpose(b)
                out_tiles(u, start=True)
            return carry

        lax.fori_loop(0, N_UNIT // 2, group, 0)
        out_tiles(N_UNIT - 1, start=False)

    return emb


def kernel(x, weights):
    b, l = x.shape
    flat_idx = x.reshape(b * l).astype(jnp.int32)
    emb = _make_emb_kernel()
    out5 = emb(flat_idx, weights)
    return jnp.transpose(out5, (2, 4, 0, 1, 3)).reshape(b, l, D_MODEL)


# speedup vs baseline: 3.9871x; 1.2786x over previous
import functools

import jax
import jax.numpy as jnp
from jax import lax
from jax.experimental import pallas as pl
from jax.experimental.pallas import tpu as pltpu
from jax.experimental.pallas import tpu_sc as plsc

D_MODEL = 32
NUM_CORES = 2
NUM_SUBCORES = 16
NUM_WORKERS = NUM_CORES * NUM_SUBCORES

B = 16384
L = 50
BT = 128
N_BT = B // BT
BT_PER_W = N_BT // NUM_WORKERS
L_GRP = 5
N_CHUNK = L // L_GRP
CHUNK_ROWS = L_GRP * BT
N_SLAB = D_MODEL // 8
N_UNIT = BT_PER_W * N_CHUNK
TPITCH = 129


VOCAB = 1000000
NTCOL = VOCAB // 128


@functools.lru_cache(maxsize=None)
def _make_format_kernel():
    mesh = plsc.VectorSubcoreMesh(core_axis_name="c", subcore_axis_name="s")

    @functools.partial(
        pl.kernel,
        mesh=mesh,
        out_type=jax.ShapeDtypeStruct((VOCAB // 4, 128), jnp.float32),
        scratch_types=[
            pltpu.VMEM((32, 128), jnp.float32),
            pltpu.VMEM((32, 128), jnp.float32),
            pltpu.SemaphoreType.DMA,
        ],
        compiler_params=pltpu.CompilerParams(
            use_tc_tiling_on_sc=True, needs_layout_passes=False),
    )
    def fmt(wt_hbm, wtail_hbm, out_hbm, tile32, blk, osem):
        wid = lax.axis_index("s") * NUM_CORES + lax.axis_index("c")
        iota16 = lax.iota(jnp.int32, 16)

        def transpose_tile():
            @plsc.parallel_loop(0, 256, unroll=8)
            def _(g):
                k = g & 15
                w0 = ((g >> 4) & 1) * 16
                c0 = (g >> 5) * 16
                ik = (iota16 + k) & 15
                wv = ik + w0
                cv = iota16 + c0
                v = plsc.load_gather(tile32, [wv, cv])
                plsc.store_scatter(blk, [cv >> 2, ((cv & 3) << 5) + wv], v)

        def do_block(j):
            pltpu.sync_copy(wt_hbm.at[:, pl.ds(j * 128, 128)], tile32)
            transpose_tile()
            pltpu.sync_copy(blk, out_hbm.at[pl.ds(j * 32, 32)])

        def body(m, carry):
            do_block(wid + m * NUM_WORKERS)
            return carry

        lax.fori_loop(0, NTCOL // NUM_WORKERS, body, 0)

        @pl.when(wid < NTCOL % NUM_WORKERS)
        def _():
            do_block((NTCOL // NUM_WORKERS) * NUM_WORKERS + wid)

        @pl.when(wid == NUM_WORKERS - 1)
        def _():
            pltpu.sync_copy(wtail_hbm, tile32)
            transpose_tile()
            pltpu.sync_copy(blk.at[pl.ds(0, 16)],
                            out_hbm.at[pl.ds(NTCOL * 32, 16)])

    return fmt


@functools.lru_cache(maxsize=None)
def _make_emb_kernel():
    mesh = plsc.VectorSubcoreMesh(core_axis_name="c", subcore_axis_name="s")

    @functools.partial(
        pl.kernel,
        mesh=mesh,
        out_type=jax.ShapeDtypeStruct((L, N_SLAB, N_BT, 8, BT), jnp.float32),
        scratch_types=[
            pltpu.VMEM((BT * L,), jnp.int32),
            pltpu.VMEM((CHUNK_ROWS,), jnp.int32),
            pltpu.VMEM((CHUNK_ROWS,), jnp.int32),
            pltpu.VMEM((CHUNK_ROWS, D_MODEL), jnp.float32),
            pltpu.VMEM((CHUNK_ROWS, D_MODEL), jnp.float32),
            pltpu.VMEM((L_GRP * D_MODEL, TPITCH), jnp.float32),
            pltpu.SemaphoreType.DMA,
            pltpu.SemaphoreType.DMA,
            pltpu.SemaphoreType.DMA,
        ],
        compiler_params=pltpu.CompilerParams(
            use_tc_tiling_on_sc=False, needs_layout_passes=False),
    )
    def emb(idx_hbm, table_hbm, out_hbm, xb, idxp0, idxp1, rows0, rows1,
            tbuf, gsem0, gsem1, osem):
        wid = lax.axis_index("s") * NUM_CORES + lax.axis_index("c")
        iota16 = lax.iota(jnp.int32, 16)
        iota_l = iota16 * L
        zeros16 = jnp.full((16,), 0, jnp.int32)
        idxp = (idxp0, idxp1)
        rows = (rows0, rows1)
        gsem = (gsem0, gsem1)

        def stage_x(u):
            bt_abs = wid * BT_PER_W + u // N_CHUNK
            pltpu.sync_copy(idx_hbm.at[pl.ds(bt_abs * (BT * L), BT * L)], xb)

        def idx_build(u, slot):
            l0 = (u % N_CHUNK) * L_GRP

            @plsc.parallel_loop(0, CHUNK_ROWS // 16, unroll=8)
            def _(g):
                ll = g >> 3
                c0 = (g & 7) * 16
                src = iota_l + (c0 * L + l0 + ll)
                idxp[slot][pl.ds(g * 16, 16)] = plsc.load_gather(xb, [src])

        def gather_start(slot):
            pltpu.async_copy(table_hbm.at[idxp[slot]], rows[slot], gsem[slot])

        def gather_wait(slot):
            pltpu.make_async_copy(
                table_hbm.at[idxp[slot]], rows[slot], gsem[slot]).wait()

        def transpose(slot):
            @plsc.parallel_loop(0, CHUNK_ROWS * 2, unroll=8)
            def _(g):
                rp = g >> 1
                half = g & 1
                ll = rp >> 7
                c = rp & 127
                v = rows[slot][rp, pl.ds(half * 16, 16)]
                rowvec = iota16 + (ll * D_MODEL + half * 16)
                colvec = zeros16 + c
                plsc.store_scatter(tbuf, [rowvec, colvec], v)

        def out_tiles(u, start):
            bt_abs = wid * BT_PER_W + u // N_CHUNK
            l0 = (u % N_CHUNK) * L_GRP
            for ll in range(L_GRP):
                for s in range(N_SLAB):
                    src = tbuf.at[pl.ds(ll * D_MODEL + s * 8, 8),
                                  pl.ds(0, BT)]
                    dst = out_hbm.at[l0 + ll, s, bt_abs]
                    if start:
                        pltpu.async_copy(src, dst, osem)
                    else:
                        pltpu.make_async_copy(src, dst, osem).wait()

        stage_x(0)
        idx_build(0, 0)
        gather_start(0)

        def group(gidx, carry):
            for b in range(2):
                o = 1 - b
                u = gidx * 2 + b

                @pl.when(jnp.logical_and(u < N_UNIT - 1,
                                         (u + 1) % N_CHUNK == 0))
                def _():
                    stage_x(u + 1)

                gather_wait(b)

                @pl.when(u < N_UNIT - 1)
                def _():
                    idx_build(u + 1, o)
                    gather_start(o)

                @pl.when(u >= 1)
                def _():
                    out_tiles(u - 1, start=False)

                transpose(b)
                out_tiles(u, start=True)
            return carry

        lax.fori_loop(0, N_UNIT // 2, group, 0)
        out_tiles(N_UNIT - 1, start=False)

    return emb


def kernel(x, weights):
    b, l = x.shape
    flat_idx = x.reshape(b * l).astype(jnp.int32)
    fmt = _make_format_kernel()
    wtail = jnp.pad(weights[NTCOL * 128:].T, ((0, 0), (0, 64)))
    w_lin = fmt(weights.T, wtail).reshape(VOCAB, D_MODEL)
    emb = _make_emb_kernel()
    out5 = emb(flat_idx, w_lin)
    return jnp.transpose(out5, (2, 4, 0, 1, 3)).reshape(b, l, D_MODEL)


# speedup vs baseline: 6.6336x; 1.6638x over previous
import functools

import jax
import jax.numpy as jnp
from jax import lax
from jax.experimental import pallas as pl
from jax.experimental.pallas import tpu as pltpu
from jax.experimental.pallas import tpu_sc as plsc

D_MODEL = 32
NUM_CORES = 2
NUM_SUBCORES = 16
NUM_WORKERS = NUM_CORES * NUM_SUBCORES

B = 16384
L = 50
BT = 128
N_BT = B // BT
BT_PER_W = N_BT // NUM_WORKERS
L_GRP = 5
N_CHUNK = L // L_GRP
CHUNK_ROWS = L_GRP * BT
N_SLAB = D_MODEL // 8
N_UNIT = BT_PER_W * N_CHUNK
TPITCH = 129


VOCAB = 1000000
NTCOL = VOCAB // 128


@functools.lru_cache(maxsize=None)
def _make_format_kernel():
    mesh = plsc.VectorSubcoreMesh(core_axis_name="c", subcore_axis_name="s")

    @functools.partial(
        pl.kernel,
        mesh=mesh,
        out_type=jax.ShapeDtypeStruct((VOCAB // 4, 128), jnp.float32),
        scratch_types=[
            pltpu.VMEM((32, 128), jnp.float32),
            pltpu.VMEM((32, 128), jnp.float32),
            pltpu.VMEM((32, 128), jnp.float32),
            pltpu.VMEM((32, 128), jnp.float32),
            pltpu.SemaphoreType.DMA,
            pltpu.SemaphoreType.DMA,
            pltpu.SemaphoreType.DMA,
            pltpu.SemaphoreType.DMA,
        ],
        compiler_params=pltpu.CompilerParams(
            use_tc_tiling_on_sc=True, needs_layout_passes=False),
    )
    def fmt(wt_hbm, wtail_hbm, out_hbm, tile0, tile1, blk0, blk1,
            isem0, isem1, osem0, osem1):
        wid = lax.axis_index("s") * NUM_CORES + lax.axis_index("c")
        iota16 = lax.iota(jnp.int32, 16)
        tile = (tile0, tile1)
        blk = (blk0, blk1)
        isem = (isem0, isem1)
        osem = (osem0, osem1)
        n_main = NTCOL // NUM_WORKERS

        def jcol(m):
            return wid + m * NUM_WORKERS

        def transpose_tile(t, k_buf):
            @plsc.parallel_loop(0, 256, unroll=8)
            def _(g):
                k = g & 15
                w0 = ((g >> 4) & 1) * 16
                c0 = (g >> 5) * 16
                ik = (iota16 + k) & 15
                wv = ik + w0
                cv = iota16 + c0
                v = plsc.load_gather(t, [wv, cv])
                plsc.store_scatter(k_buf, [cv >> 2, ((cv & 3) << 5) + wv], v)

        def in_start(m, b):
            pltpu.async_copy(wt_hbm.at[:, pl.ds(jcol(m) * 128, 128)],
                             tile[b], isem[b])

        def in_wait(m, b):
            pltpu.make_async_copy(wt_hbm.at[:, pl.ds(jcol(m) * 128, 128)],
                                  tile[b], isem[b]).wait()

        def out_start(m, b):
            pltpu.async_copy(blk[b], out_hbm.at[pl.ds(jcol(m) * 32, 32)],
                             osem[b])

        def out_wait(m, b):
            pltpu.make_async_copy(blk[b],
                                  out_hbm.at[pl.ds(jcol(m) * 32, 32)],
                                  osem[b]).wait()

        in_start(0, 0)

        def group(g, carry):
            for b in range(2):
                o = 1 - b
                m = g * 2 + b

                @pl.when(m < n_main - 1)
                def _():
                    in_start(m + 1, o)

                in_wait(m, b)

                @pl.when(m >= 2)
                def _():
                    out_wait(m - 2, b)

                transpose_tile(tile[b], blk[b])
                out_start(m, b)
            return carry

        lax.fori_loop(0, n_main // 2, group, 0)
        out_wait(n_main - 2, 0)
        out_wait(n_main - 1, 1)

        @pl.when(wid < NTCOL % NUM_WORKERS)
        def _():
            j = n_main * NUM_WORKERS + wid
            pltpu.sync_copy(wt_hbm.at[:, pl.ds(j * 128, 128)], tile0)
            transpose_tile(tile0, blk0)
            pltpu.sync_copy(blk0, out_hbm.at[pl.ds(j * 32, 32)])

        @pl.when(wid == NUM_WORKERS - 1)
        def _():
            pltpu.sync_copy(wtail_hbm, tile0)
            transpose_tile(tile0, blk0)
            pltpu.sync_copy(blk0.at[pl.ds(0, 16)],
                            out_hbm.at[pl.ds(NTCOL * 32, 16)])

    return fmt


@functools.lru_cache(maxsize=None)
def _make_emb_kernel():
    mesh = plsc.VectorSubcoreMesh(core_axis_name="c", subcore_axis_name="s")

    @functools.partial(
        pl.kernel,
        mesh=mesh,
        out_type=jax.ShapeDtypeStruct((L, N_SLAB, N_BT, 8, BT), jnp.float32),
        scratch_types=[
            pltpu.VMEM((BT * L,), jnp.int32),
            pltpu.VMEM((CHUNK_ROWS,), jnp.int32),
            pltpu.VMEM((CHUNK_ROWS,), jnp.int32),
            pltpu.VMEM((CHUNK_ROWS, D_MODEL), jnp.float32),
            pltpu.VMEM((CHUNK_ROWS, D_MODEL), jnp.float32),
            pltpu.VMEM((L_GRP * D_MODEL, TPITCH), jnp.float32),
            pltpu.SemaphoreType.DMA,
            pltpu.SemaphoreType.DMA,
            pltpu.SemaphoreType.DMA,
        ],
        compiler_params=pltpu.CompilerParams(
            use_tc_tiling_on_sc=False, needs_layout_passes=False),
    )
    def emb(idx_hbm, table_hbm, out_hbm, xb, idxp0, idxp1, rows0, rows1,
            tbuf, gsem0, gsem1, osem):
        wid = lax.axis_index("s") * NUM_CORES + lax.axis_index("c")
        iota16 = lax.iota(jnp.int32, 16)
        iota_l = iota16 * L
        zeros16 = jnp.full((16,), 0, jnp.int32)
        idxp = (idxp0, idxp1)
        rows = (rows0, rows1)
        gsem = (gsem0, gsem1)

        def stage_x(u):
            bt_abs = wid * BT_PER_W + u // N_CHUNK
            pltpu.sync_copy(idx_hbm.at[pl.ds(bt_abs * (BT * L), BT * L)], xb)

        def idx_build(u, slot):
            l0 = (u % N_CHUNK) * L_GRP

            @plsc.parallel_loop(0, CHUNK_ROWS // 16, unroll=8)
            def _(g):
                ll = g >> 3
                c0 = (g & 7) * 16
                src = iota_l + (c0 * L + l0 + ll)
                idxp[slot][pl.ds(g * 16, 16)] = plsc.load_gather(xb, [src])

        def gather_start(slot):
            pltpu.async_copy(table_hbm.at[idxp[slot]], rows[slot], gsem[slot])

        def gather_wait(slot):
            pltpu.make_async_copy(
                table_hbm.at[idxp[slot]], rows[slot], gsem[slot]).wait()

        def transpose(slot):
            @plsc.parallel_loop(0, CHUNK_ROWS * 2, unroll=8)
            def _(g):
                rp = g >> 1
                half = g & 1
                ll = rp >> 7
                c = rp & 127
                v = rows[slot][rp, pl.ds(half * 16, 16)]
                rowvec = iota16 + (ll * D_MODEL + half * 16)
                colvec = zeros16 + c
                plsc.store_scatter(tbuf, [rowvec, colvec], v)

        def out_tiles(u, start):
            bt_abs = wid * BT_PER_W + u // N_CHUNK
            l0 = (u % N_CHUNK) * L_GRP
            for ll in range(L_GRP):
                for s in range(N_SLAB):
                    src = tbuf.at[pl.ds(ll * D_MODEL + s * 8, 8),
                                  pl.ds(0, BT)]
                    dst = out_hbm.at[l0 + ll, s, bt_abs]
                    if start:
                        pltpu.async_copy(src, dst, osem)
                    else:
                        pltpu.make_async_copy(src, dst, osem).wait()

        stage_x(0)
        idx_build(0, 0)
        gather_start(0)

        def group(gidx, carry):
            for b in range(2):
                o = 1 - b
                u = gidx * 2 + b

                @pl.when(jnp.logical_and(u < N_UNIT - 1,
                                         (u + 1) % N_CHUNK == 0))
                def _():
                    stage_x(u + 1)

                gather_wait(b)

                @pl.when(u < N_UNIT - 1)
                def _():
                    idx_build(u + 1, o)
                    gather_start(o)

                @pl.when(u >= 1)
                def _():
                    out_tiles(u - 1, start=False)

                transpose(b)
                out_tiles(u, start=True)
            return carry

        lax.fori_loop(0, N_UNIT // 2, group, 0)
        out_tiles(N_UNIT - 1, start=False)

    return emb


def kernel(x, weights):
    b, l = x.shape
    flat_idx = x.reshape(b * l).astype(jnp.int32)
    fmt = _make_format_kernel()
    wtail = jnp.pad(weights[NTCOL * 128:].T, ((0, 0), (0, 64)))
    w_lin = fmt(weights.T, wtail).reshape(VOCAB, D_MODEL)
    emb = _make_emb_kernel()
    out5 = emb(flat_idx, w_lin)
    return jnp.transpose(out5, (2, 4, 0, 1, 3)).reshape(b, l, D_MODEL)
